# Initial kernel scaffold; baseline (speedup 1.0000x reference)
import jax
import jax.numpy as jnp
from jax.experimental import pallas as pl


def kernel(x, edge_index, pos, batch, atom_W, atom_b, msg_W1, msg_b1, msg_W2, msg_b2, upd_W, upd_b, ln_g, ln_b, head_W1, head_b1, head_W2, head_b2, head_W3, head_b3):
    raise NotImplementedError("write your pallas kernel here")


# speedup vs baseline: 5.2559x; 5.2559x over previous
import functools

import jax
import jax.numpy as jnp
from jax import lax
from jax.experimental import pallas as pl
from jax.experimental.pallas import tpu as pltpu
from jax.experimental.pallas import tpu_sc as plsc

N = 50000
E = 800000
D_IN = 128
H = 64
HH = 32
L = 3
G = 64

R = 1000
GRID = N // R

CH = 128
NCHUNK = E // CH
NS = 16
NC = 2
RPT = 3128
LAST_ROWS = N - (NS - 1) * RPT

_EPS = 1e-5


def _zero_sp(zb, sp, start, rows):
    nz = zb.shape[0]
    q, rem = divmod(rows, nz)

    def dz(k, _):
        pltpu.sync_copy(zb, sp.at[pl.ds(start + k * nz, nz)])
        return _
    lax.fori_loop(0, q, dz, None)
    if rem:
        pltpu.sync_copy(zb.at[pl.ds(0, rem)],
                        sp.at[pl.ds(start + q * nz, rem)])


def _sp_to_hbm(sp, dref, stage, src_start, dst_start, rows):
    ns = stage.shape[0]
    q, rem = divmod(rows, ns)

    def mv(k, _):
        pltpu.sync_copy(sp.at[pl.ds(src_start + k * ns, ns)], stage)
        pltpu.sync_copy(stage, dref.at[pl.ds(dst_start + k * ns, ns)])
        return _
    lax.fori_loop(0, q, mv, None)
    if rem:
        st = stage.at[pl.ds(0, rem)]
        pltpu.sync_copy(sp.at[pl.ds(src_start + q * ns, rem)], st)
        pltpu.sync_copy(st, dref.at[pl.ds(dst_start + q * ns, rem)])


def _per_tile_rows(sid, fn):
    @pl.when(sid < NS - 1)
    def _():
        fn(sid * RPT, RPT)

    @pl.when(sid == NS - 1)
    def _():
        fn((NS - 1) * RPT, LAST_ROWS)


def _q16(v):
    u = plsc.bitcast(v, jnp.int32)
    r = (u + 0x7FFF + ((u >> 16) & 1)) & jnp.int32(-65536)
    return plsc.bitcast(r, jnp.float32)


def _chunk_base_count(wid, nworkers):
    per = NCHUNK // nworkers
    extra = NCHUNK - per * nworkers
    base = wid * per + jnp.minimum(wid, extra)
    cnt = per + jnp.where(wid < extra, 1, 0)
    return base, cnt, per + (1 if extra > 0 else 0)


CMAX_D2 = NCHUNK // (NC * NS) + 1


def _sc_d2_cnt_body(posx, posy, posz, dst1d, src1d, d2_out, cnt2,
                    dstrow, srcrow, table, d2acc, ones_b, zb, cnt_sp):
    cid = lax.axis_index("c")
    sid = lax.axis_index("s")
    wid = cid * NS + sid

    def zloop(r, _):
        zb[pl.ds(r * 16, 16)] = jnp.zeros((16,), jnp.float32)
        return _
    lax.fori_loop(0, RPT // 16 + 1, zloop, None)
    for r in range(CH // 16):
        ones_b[pl.ds(r * 16, 16)] = jnp.ones((16,), jnp.float32)

    def zr(start, rows):
        pltpu.sync_copy(zb.at[pl.ds(0, rows)], cnt_sp.at[pl.ds(start, rows)])
    _per_tile_rows(sid, zr)
    plsc.subcore_barrier()

    base, mychunks, _ = _chunk_base_count(wid, NC * NS)

    for coord, pref in enumerate((posx, posy, posz)):
        pltpu.sync_copy(pref, table)

        def chunk(c, _):
            @pl.when(c < mychunks)
            def _():
                g = base + c
                pltpu.sync_copy(dst1d.at[pl.ds(g * CH, CH)], dstrow)
                pltpu.sync_copy(src1d.at[pl.ds(g * CH, CH)], srcrow)

                def grp(j, _):
                    js = pl.ds(j * 16, 16)
                    xs = plsc.load_gather(table, [srcrow[js]])
                    xd = plsc.load_gather(table, [dstrow[js]])
                    dv = xs - xd
                    ds = pl.ds(c * CH + j * 16, 16)
                    if coord == 0:
                        d2acc[ds] = dv * dv
                    else:
                        d2acc[ds] = d2acc[ds] + dv * dv
                    return _
                lax.fori_loop(0, CH // 16, grp, None)
                if coord == 0:
                    pltpu.sync_copy(ones_b, cnt_sp.at[dstrow], add=True)
            return _
        lax.fori_loop(0, CMAX_D2, chunk, None)

    def wchunk(c, _):
        @pl.when(c < mychunks)
        def _():
            pltpu.sync_copy(d2acc.at[pl.ds(c * CH, CH)],
                            d2_out.at[pl.ds((base + c) * CH, CH)])
        return _
    lax.fori_loop(0, CMAX_D2, wchunk, None)

    plsc.subcore_barrier()

    def go(start, rows):
        _sp_to_hbm(cnt_sp, cnt2, d2acc, start, cid * N + start, rows)
    _per_tile_rows(sid, go)


def _make_sc_d2_cnt():
    mesh = plsc.VectorSubcoreMesh(core_axis_name="c", subcore_axis_name="s")
    return pl.kernel(
        _sc_d2_cnt_body,
        out_type=(
            jax.ShapeDtypeStruct((E,), jnp.float32),
            jax.ShapeDtypeStruct((2 * N,), jnp.float32),
        ),
        mesh=mesh,
        compiler_params=pltpu.CompilerParams(needs_layout_passes=False),
        scratch_types=[
            pltpu.VMEM((CH,), jnp.int32),
            pltpu.VMEM((CH,), jnp.int32),
            pltpu.VMEM((N,), jnp.float32),
            pltpu.VMEM((CMAX_D2 * CH,), jnp.float32),
            pltpu.VMEM((CH,), jnp.float32),
            pltpu.VMEM((3136,), jnp.float32),
            pltpu.VMEM_SHARED((N,), jnp.float32),
        ],
    )


def _sc_edge_body(a2, b2, whbm, dst1d, src1d, d2_1d, s2_out,
                  dstA, srcA, gdstA, gsrcA, d2A, arA, brA,
                  dstB, srcB, gdstB, gsrcB, d2B, arB, brB,
                  wv, zb, sbuf, s_sp,
                  semIA, semIB, semGA, semGB, semSA, semSB):
    cid = lax.axis_index("c")
    sid = lax.axis_index("s")
    goff = cid * N

    def zloop(r, _):
        zb[r, pl.ds(0, 16)] = jnp.zeros((16,), jnp.float32)
        zb[r, pl.ds(16, 16)] = jnp.zeros((16,), jnp.float32)
        return _
    lax.fori_loop(0, zb.shape[0], zloop, None)
    _per_tile_rows(sid, lambda start, rows: _zero_sp(zb, s_sp, start, rows))
    pltpu.sync_copy(whbm.at[pl.ds(cid * HH, HH)], wv)
    plsc.subcore_barrier()

    base, mychunks, cmax = _chunk_base_count(sid, NS)
    w0 = wv[pl.ds(0, 16)]
    w1 = wv[pl.ds(16, 16)]

    def idx_start(g, dstr, srcr, d2r, semI):
        pltpu.async_copy(dst1d.at[pl.ds(g * CH, CH)], dstr, semI)
        pltpu.async_copy(src1d.at[pl.ds(g * CH, CH)], srcr, semI)
        pltpu.async_copy(d2_1d.at[pl.ds(g * CH, CH)],
                         d2r.at[pl.ds(0, CH)], semI)

    def idx_wait(g, dstr, srcr, d2r, semI):
        pltpu.make_async_copy(dst1d.at[pl.ds(g * CH, CH)], dstr, semI).wait()
        pltpu.make_async_copy(src1d.at[pl.ds(g * CH, CH)], srcr, semI).wait()
        pltpu.make_async_copy(d2_1d.at[pl.ds(g * CH, CH)],
                              d2r.at[pl.ds(0, CH)], semI).wait()

    def gidx(dstr, srcr, gdst, gsrc):
        def gj(j, _):
            js = pl.ds(j * 16, 16)
            gdst[js] = dstr[js] + goff
            gsrc[js] = srcr[js] + goff
            return _
        lax.fori_loop(0, CH // 16, gj, None)

    def gather_start(gdst, gsrc, ar, br, semG):
        pltpu.async_copy(a2.at[gdst], ar, semG)
        pltpu.async_copy(b2.at[gsrc], br, semG)

    def gather_wait(gdst, gsrc, ar, br, semG):
        pltpu.make_async_copy(a2.at[gdst], ar, semG).wait()
        pltpu.make_async_copy(b2.at[gsrc], br, semG).wait()

    def scat_start(dstr, ar, semS):
        pltpu.async_copy(ar, s_sp.at[dstr], semS, add=True)

    def scat_wait(dstr, ar, semS):
        pltpu.make_async_copy(ar, s_sp.at[dstr], semS).wait()

    def compute(d2r, ar, br):
        def edge(e, _):
            dd = d2r[pl.ds(e, 16)][0]
            t0 = jnp.maximum(ar[e, pl.ds(0, 16)]
                             + br[e, pl.ds(0, 16)] + dd * w0, 0.0)
            t1 = jnp.maximum(ar[e, pl.ds(16, 16)]
                             + br[e, pl.ds(16, 16)] + dd * w1, 0.0)
            ar[e, pl.ds(0, 16)] = t0
            ar[e, pl.ds(16, 16)] = t1
            return _
        lax.fori_loop(0, CH, edge, None)

    bufs = ((dstA, srcA, gdstA, gsrcA, d2A, arA, brA, semIA, semGA, semSA),
            (dstB, srcB, gdstB, gsrcB, d2B, arB, brB, semIB, semGB, semSB))

    @pl.when(mychunks > 0)
    def _():
        dstr, srcr, gdst, gsrc, d2r, ar, br, semI, semG, semS = bufs[0]
        pltpu.sync_copy(dst1d.at[pl.ds(base * CH, CH)], dstr)
        pltpu.sync_copy(src1d.at[pl.ds(base * CH, CH)], srcr)
        pltpu.sync_copy(d2_1d.at[pl.ds(base * CH, CH)], d2r.at[pl.ds(0, CH)])
        gidx(dstr, srcr, gdst, gsrc)
        gather_start(gdst, gsrc, ar, br, semG)

    def step(c, cur, nxt):
        dstr, srcr, gdst, gsrc, d2r, ar, br, semI, semG, semS = cur
        ndstr, nsrcr, ngdst, ngsrc, nd2r, nar, nbr, nsemI, nsemG, nsemS = nxt
        g = base + c

        @pl.when(c >= 1)
        def _():
            scat_wait(ndstr, nar, nsemS)

        @pl.when(c + 1 < mychunks)
        def _():
            idx_start(g + 1, ndstr, nsrcr, nd2r, nsemI)

        gather_wait(gdst, gsrc, ar, br, semG)
        compute(d2r, ar, br)

        @pl.when(c + 1 < mychunks)
        def _():
            idx_wait(g + 1, ndstr, nsrcr, nd2r, nsemI)
            gidx(ndstr, nsrcr, ngdst, ngsrc)
            gather_start(ngdst, ngsrc, nar, nbr, nsemG)

        scat_start(dstr, ar, semS)

    def chunk(c, _):
        @pl.when(c < mychunks)
        def _():
            @pl.when(c % 2 == 0)
            def _():
                step(c, bufs[0], bufs[1])

            @pl.when(c % 2 == 1)
            def _():
                step(c, bufs[1], bufs[0])
        return _
    lax.fori_loop(0, cmax, chunk, None)

    @pl.when(jnp.logical_and(mychunks > 0, (mychunks - 1) % 2 == 0))
    def _():
        scat_wait(bufs[0][0], bufs[0][5], bufs[0][9])

    @pl.when(jnp.logical_and(mychunks > 0, (mychunks - 1) % 2 == 1))
    def _():
        scat_wait(bufs[1][0], bufs[1][5], bufs[1][9])

    plsc.subcore_barrier()

    def go(start, rows):
        _sp_to_hbm(s_sp, s2_out, sbuf, start, cid * N + start, rows)
    _per_tile_rows(sid, go)


def _make_sc_edge():
    mesh = plsc.VectorSubcoreMesh(core_axis_name="c", subcore_axis_name="s")
    return pl.kernel(
        _sc_edge_body,
        out_type=jax.ShapeDtypeStruct((2 * N, HH), jnp.float32),
        mesh=mesh,
        compiler_params=pltpu.CompilerParams(use_tc_tiling_on_sc=False,
                                             needs_layout_passes=False),
        scratch_types=[
            pltpu.VMEM((CH,), jnp.int32),
            pltpu.VMEM((CH,), jnp.int32),
            pltpu.VMEM((CH,), jnp.int32),
            pltpu.VMEM((CH,), jnp.int32),
            pltpu.VMEM((CH + 16,), jnp.float32),
            pltpu.VMEM((CH, HH), jnp.float32),
            pltpu.VMEM((CH, HH), jnp.float32),
            pltpu.VMEM((CH,), jnp.int32),
            pltpu.VMEM((CH,), jnp.int32),
            pltpu.VMEM((CH,), jnp.int32),
            pltpu.VMEM((CH,), jnp.int32),
            pltpu.VMEM((CH + 16,), jnp.float32),
            pltpu.VMEM((CH, HH), jnp.float32),
            pltpu.VMEM((CH, HH), jnp.float32),
            pltpu.VMEM((HH,), jnp.float32),
            pltpu.VMEM((64, HH), jnp.float32),
            pltpu.VMEM((120, HH), jnp.float32),
            pltpu.VMEM_SHARED((N, HH), jnp.float32),
            pltpu.SemaphoreType.DMA,
            pltpu.SemaphoreType.DMA,
            pltpu.SemaphoreType.DMA,
            pltpu.SemaphoreType.DMA,
            pltpu.SemaphoreType.DMA,
            pltpu.SemaphoreType.DMA,
        ],
    )


def _hdot(a, b):
    return jnp.dot(a, b, preferred_element_type=jnp.float32)


def _tc_embed_body(x, atom_W, atom_b, w1d, w1s, b1,
                   h_out, a2_out, b2_out):
    h = _hdot(x[...], atom_W[...]) + atom_b[...]
    h_out[...] = h
    a2_out[...] = _hdot(h, w1d[0]) + b1[0]
    b2_out[...] = _hdot(h, w1s[0])


def _make_tc_embed():
    return pl.pallas_call(
        _tc_embed_body,
        grid=(NC, GRID),
        in_specs=[
            pl.BlockSpec((R, D_IN), lambda c, i: (i, 0)),
            pl.BlockSpec((D_IN, H), lambda c, i: (0, 0)),
            pl.BlockSpec((1, H), lambda c, i: (0, 0)),
            pl.BlockSpec((1, H, HH), lambda c, i: (c, 0, 0)),
            pl.BlockSpec((1, H, HH), lambda c, i: (c, 0, 0)),
            pl.BlockSpec((1, 1, HH), lambda c, i: (c, 0, 0)),
        ],
        out_specs=[
            pl.BlockSpec((R, H), lambda c, i: (i, 0)),
            pl.BlockSpec((R, HH), lambda c, i: (c * GRID + i, 0)),
            pl.BlockSpec((R, HH), lambda c, i: (c * GRID + i, 0)),
        ],
        out_shape=[
            jax.ShapeDtypeStruct((N, H), jnp.float32),
            jax.ShapeDtypeStruct((2 * N, HH), jnp.float32),
            jax.ShapeDtypeStruct((2 * N, HH), jnp.float32),
        ],
    )


def _tc_update_body(last, h, s0, s1, cnt0, cnt1, W2, b2, updW, updb,
                    lng, lnb, w1d, w1s, b1, *outs):
    cnt = cnt0[...] + cnt1[...]
    agg = (_hdot(s0[...], W2[0:HH, :]) + _hdot(s1[...], W2[HH:H, :])
           + cnt * b2[...])
    hv = h[...]
    u = (_hdot(hv, updW[0:H, :]) + _hdot(agg, updW[H:2 * H, :])
         + updb[...])
    mu = jnp.mean(u, axis=-1, keepdims=True)
    var = jnp.mean((u - mu) ** 2, axis=-1, keepdims=True)
    v1 = var + _EPS
    r0 = lax.rsqrt(v1)
    rinv = r0 * (1.5 - 0.5 * v1 * r0 * r0)
    un = (u - mu) * rinv * lng[...] + lnb[...]
    hn = jnp.maximum(hv + un, 0.0)
    outs[0][...] = hn
    if not last:
        outs[1][...] = _hdot(hn, w1d[0]) + b1[0]
        outs[2][...] = _hdot(hn, w1s[0])


def _make_tc_update(last):
    if last:
        grid = (1, GRID)
        out_specs = [pl.BlockSpec((R, H), lambda c, i: (i, 0))]
        out_shape = [jax.ShapeDtypeStruct((N, H), jnp.float32)]
    else:
        grid = (NC, GRID)
        out_specs = [
            pl.BlockSpec((R, H), lambda c, i: (i, 0)),
            pl.BlockSpec((R, HH), lambda c, i: (c * GRID + i, 0)),
            pl.BlockSpec((R, HH), lambda c, i: (c * GRID + i, 0)),
        ]
        out_shape = [
            jax.ShapeDtypeStruct((N, H), jnp.float32),
            jax.ShapeDtypeStruct((2 * N, HH), jnp.float32),
            jax.ShapeDtypeStruct((2 * N, HH), jnp.float32),
        ]
    return pl.pallas_call(
        functools.partial(_tc_update_body, last),
        grid=grid,
        in_specs=[
            pl.BlockSpec((R, H), lambda c, i: (i, 0)),
            pl.BlockSpec((R, HH), lambda c, i: (i, 0)),
            pl.BlockSpec((R, HH), lambda c, i: (GRID + i, 0)),
            pl.BlockSpec((R, 1), lambda c, i: (i, 0)),
            pl.BlockSpec((R, 1), lambda c, i: (GRID + i, 0)),
            pl.BlockSpec((H, H), lambda c, i: (0, 0)),
            pl.BlockSpec((1, H), lambda c, i: (0, 0)),
            pl.BlockSpec((2 * H, H), lambda c, i: (0, 0)),
            pl.BlockSpec((1, H), lambda c, i: (0, 0)),
            pl.BlockSpec((1, H), lambda c, i: (0, 0)),
            pl.BlockSpec((1, H), lambda c, i: (0, 0)),
            pl.BlockSpec((1, H, HH), lambda c, i: (c, 0, 0)),
            pl.BlockSpec((1, H, HH), lambda c, i: (c, 0, 0)),
            pl.BlockSpec((1, 1, HH), lambda c, i: (c, 0, 0)),
        ],
        out_specs=out_specs,
        out_shape=out_shape,
    )


def _tc_pool_head_body(batch, h, hW1, hb1, hW2, hb2, hW3, hb3,
                       out, sums, cnts):
    i = pl.program_id(0)
    bb = batch[...]
    iot = lax.broadcasted_iota(jnp.int32, (R, G), 1)
    maskT = (bb == iot).astype(jnp.float32)
    part = lax.dot_general(maskT, h[...], (((0,), (0,)), ((), ())),
                           preferred_element_type=jnp.float32,
                           precision=lax.Precision.HIGHEST)
    ones = jnp.ones((R, 1), jnp.float32)
    pcnt = lax.dot_general(maskT, ones, (((0,), (0,)), ((), ())),
                           preferred_element_type=jnp.float32,
                           precision=lax.Precision.HIGHEST)


    @pl.when(i == 0)
    def _():
        sums[...] = part
        cnts[...] = pcnt

    @pl.when(i > 0)
    def _():
        sums[...] = sums[...] + part
        cnts[...] = cnts[...] + pcnt

    @pl.when(i == GRID - 1)
    def _():
        c = jnp.maximum(cnts[...], 1.0)
        rc0 = 1.0 / c
        rc = rc0 * (2.0 - c * rc0)
        gr = sums[...] * rc
        z = jnp.maximum(_hdot(gr, hW1[...]) + hb1[...], 0.0)
        z = jnp.maximum(_hdot(z, hW2[...]) + hb2[...], 0.0)
        out[...] = _hdot(z, hW3[...]) + hb3[...]


def _make_tc_pool_head():
    return pl.pallas_call(
        _tc_pool_head_body,
        grid=(GRID,),
        in_specs=[
            pl.BlockSpec((R, 1), lambda i: (i, 0)),
            pl.BlockSpec((R, H), lambda i: (i, 0)),
            pl.BlockSpec((H, H), lambda i: (0, 0)),
            pl.BlockSpec((1, H), lambda i: (0, 0)),
            pl.BlockSpec((H, H // 2), lambda i: (0, 0)),
            pl.BlockSpec((1, H // 2), lambda i: (0, 0)),
            pl.BlockSpec((H // 2, 1), lambda i: (0, 0)),
            pl.BlockSpec((1, 1), lambda i: (0, 0)),
        ],
        out_specs=pl.BlockSpec((G, 1), lambda i: (0, 0)),
        out_shape=jax.ShapeDtypeStruct((G, 1), jnp.float32),
        scratch_shapes=[
            pltpu.VMEM((G, H), jnp.float32),
            pltpu.VMEM((G, 1), jnp.float32),
        ],
    )


def kernel(x, edge_index, pos, batch, atom_W, atom_b, msg_W1, msg_b1,
           msg_W2, msg_b2, upd_W, upd_b, ln_g, ln_b, head_W1, head_b1,
           head_W2, head_b2, head_W3, head_b3):
    dst1d = edge_index[1]
    src1d = edge_index[0]
    posx, posy, posz = pos[:, 0], pos[:, 1], pos[:, 2]
    batch2d = batch.reshape(N, 1)

    w1d = [msg_W1[i, 0:H, :].reshape(H, NC, HH).transpose(1, 0, 2)
           for i in range(L)]
    w1s = [msg_W1[i, H:2 * H, :].reshape(H, NC, HH).transpose(1, 0, 2)
           for i in range(L)]
    wd2 = [msg_W1[i, 2 * H, :] for i in range(L)]
    b1r = [msg_b1[i].reshape(NC, 1, HH) for i in range(L)]
    b2r = [msg_b2[i].reshape(1, H) for i in range(L)]
    updbr = [upd_b[i].reshape(1, H) for i in range(L)]
    lngr = [ln_g[i].reshape(1, H) for i in range(L)]
    lnbr = [ln_b[i].reshape(1, H) for i in range(L)]
    atom_b2 = atom_b.reshape(1, H)
    hb1 = head_b1.reshape(1, H)
    hb2 = head_b2.reshape(1, H // 2)
    hb3 = head_b3.reshape(1, 1)

    d2_1d, cnt2 = _make_sc_d2_cnt()(posx, posy, posz, dst1d, src1d)
    cnt2 = cnt2.reshape(2 * N, 1)

    h, a2, b2v = _make_tc_embed()(x, atom_W, atom_b2, w1d[0], w1s[0], b1r[0])

    sc_edge = _make_sc_edge()
    for i in range(L):
        s2 = sc_edge(a2, b2v, wd2[i], dst1d, src1d, d2_1d)
        last = i == L - 1
        upd = _make_tc_update(last)
        nxt = 0 if last else i + 1
        outs = upd(h, s2, s2, cnt2, cnt2, msg_W2[i], b2r[i], upd_W[i],
                   updbr[i], lngr[i], lnbr[i], w1d[nxt], w1s[nxt], b1r[nxt])
        if last:
            (h,) = outs
        else:
            h, a2, b2v = outs

    pred = _make_tc_pool_head()(batch2d, h, head_W1, hb1, head_W2, hb2,
                                head_W3, hb3)
    return pred
